# Initial kernel scaffold; baseline (speedup 1.0000x reference)
import functools

import jax
import jax.numpy as jnp
from jax import lax
from jax.experimental import pallas as pl
from jax.experimental.pallas import tpu as pltpu
from jax.experimental.pallas import tpu_sc as plsc

_CATS = 26
_NCAT = 1000
_INUM = 13
_D = 64
_B = 4096
_NC = 2
_NS = 16
_NW = _NC * _NS
_BPW = _B // _NW
_BSUB = 16
_NSUB = _BPW // _BSUB
_NIDX = _BSUB * _CATS
_PIECE = 104
_NPIECE = _NIDX // _PIECE
_NUMROWS = 1 + _INUM
_TOK = _NUMROWS + _CATS


def _sc_body(xnum_hbm, xcat_hbm, w_hbm, table_hbm, bias_hbm, offs_hbm,
             out_hbm,
             xnum_v, xcat_v, offs_v, idx_v, w_v, bias_v, catstage, combuf,
             gsem):
    wid = lax.axis_index("s") * _NC + lax.axis_index("c")
    b0 = wid * _BPW

    pltpu.sync_copy(w_hbm, w_v)
    pltpu.sync_copy(bias_hbm, bias_v)
    pltpu.sync_copy(offs_hbm, offs_v)
    pltpu.sync_copy(xnum_hbm.at[pl.ds(b0 * _INUM, _BPW * _INUM)], xnum_v)

    def sub_chunk(s, carry):
        bs = b0 + s * _BSUB

        pltpu.sync_copy(xcat_hbm.at[pl.ds(bs * _CATS, _NIDX)], xcat_v)

        def idx_step(j, c):
            sl = pl.ds(j * 16, 16)
            idx_v[sl] = xcat_v[sl] + offs_v[sl]
            return c
        lax.fori_loop(0, _NIDX // 16, idx_step, 0)

        copies = [
            pltpu.async_copy(
                table_hbm.at[idx_v.at[pl.ds(p * _PIECE, _PIECE)]],
                catstage.at[pl.ds(p * _PIECE, _PIECE), :],
                gsem)
            for p in range(_NPIECE)
        ]

        for i in range(_NUMROWS):
            wr = [w_v[i, pl.ds(j * 16, 16)] for j in range(_D // 16)]
            if i == 0:
                def num_step(bb, c, wr=wr):
                    for j in range(_D // 16):
                        combuf[bb, 0, pl.ds(j * 16, 16)] = wr[j]
                    return c
            else:
                br = [bias_v[i - 1, pl.ds(j * 16, 16)]
                      for j in range(_D // 16)]

                def num_step(bb, c, i=i, wr=wr, br=br):
                    x = xnum_v[(s * _BSUB + bb) * _INUM + (i - 1)]
                    xv = jnp.full((16,), x, jnp.float32)
                    for j in range(_D // 16):
                        combuf[bb, i, pl.ds(j * 16, 16)] = wr[j] * xv + br[j]
                    return c
            lax.fori_loop(0, _BSUB, num_step, 0)

        for c in copies:
            c.wait()

        for j in range(_D // 16):
            sl = pl.ds(j * 16, 16)
            br = [bias_v[_INUM + f, sl] for f in range(_CATS)]

            def bias_step(bb, c, br=br, sl=sl):
                for f in range(_CATS):
                    combuf[bb, _NUMROWS + f, sl] = (
                        catstage[bb * _CATS + f, sl] + br[f])
                return c
            lax.fori_loop(0, _BSUB, bias_step, 0)

        pltpu.sync_copy(combuf, out_hbm.at[pl.ds(bs, _BSUB)])
        return carry

    lax.fori_loop(0, _NSUB, sub_chunk, 0)


@jax.jit
def kernel(x_num, x_cat, cls_num_weight, cat_table, bias):
    offs = jnp.tile(jnp.arange(_CATS, dtype=jnp.int32) * _NCAT, _BSUB)
    xnum_flat = x_num.reshape(-1)
    xcat_flat = x_cat.astype(jnp.int32).reshape(-1)

    mesh = plsc.VectorSubcoreMesh(core_axis_name="c", subcore_axis_name="s")
    k = pl.kernel(
        _sc_body,
        out_type=jax.ShapeDtypeStruct((_B, _TOK, _D), jnp.float32),
        mesh=mesh,
        scratch_types=[
            pltpu.VMEM((_BPW * _INUM,), jnp.float32),
            pltpu.VMEM((_NIDX,), jnp.int32),
            pltpu.VMEM((_NIDX,), jnp.int32),
            pltpu.VMEM((_NIDX,), jnp.int32),
            pltpu.VMEM((_NUMROWS, _D), jnp.float32),
            pltpu.VMEM((_INUM + _CATS, _D), jnp.float32),
            pltpu.VMEM((_NIDX, _D), jnp.float32),
            pltpu.VMEM((_BSUB, _TOK, _D), jnp.float32),
            pltpu.SemaphoreType.DMA,
        ],
    )
    return k(xnum_flat, xcat_flat, cls_num_weight, cat_table, bias, offs)


# speedup vs baseline: 2.4589x; 2.4589x over previous
import functools

import jax
import jax.numpy as jnp
from jax import lax
from jax.experimental import pallas as pl
from jax.experimental.pallas import tpu as pltpu
from jax.experimental.pallas import tpu_sc as plsc

_CATS = 26
_NCAT = 1000
_INUM = 13
_D = 64
_B = 4096
_NC = 2
_NS = 16
_NW = _NC * _NS
_BPW = _B // _NW
_BSUB = 16
_NSUB = _BPW // _BSUB
_NIDX = _BSUB * _CATS
_PIECE = 104
_NPIECE = _NIDX // _PIECE
_NUMROWS = 1 + _INUM
_TOK = _NUMROWS + _CATS


def _sc_body(xnum_hbm, xcat_hbm, w_hbm, table_hbm, bias_hbm, offs_hbm,
             out_hbm,
             xnum_v, xcat_v, offs_v, idx_v, w_v, bias_v, catstage, combuf,
             gsem):
    wid = lax.axis_index("s") * _NC + lax.axis_index("c")
    b0 = wid * _BPW

    pltpu.sync_copy(w_hbm, w_v)
    pltpu.sync_copy(bias_hbm, bias_v)
    pltpu.sync_copy(offs_hbm, offs_v)
    pltpu.sync_copy(xnum_hbm.at[pl.ds(b0 * 16, _BPW * 16)], xnum_v)

    def sub_chunk(s, carry):
        bs = b0 + s * _BSUB

        pltpu.sync_copy(xcat_hbm.at[pl.ds(bs * _CATS, _NIDX)], xcat_v)

        def idx_step(j, c):
            sl = pl.ds(j * 16, 16)
            idx_v[sl] = xcat_v[sl] + offs_v[sl]
            return c
        lax.fori_loop(0, _NIDX // 16, idx_step, 0)

        copies = [
            pltpu.async_copy(
                table_hbm.at[idx_v.at[pl.ds(p * _PIECE, _PIECE)]],
                catstage.at[pl.ds(p * _PIECE, _PIECE), :],
                gsem)
            for p in range(_NPIECE)
        ]

        for i in range(_NUMROWS):
            wr = [w_v[i, pl.ds(j * 16, 16)] for j in range(_D // 16)]
            if i == 0:
                def num_step(bb, c, wr=wr):
                    for j in range(_D // 16):
                        combuf[bb, 0, pl.ds(j * 16, 16)] = wr[j]
                    return c
            else:
                br = [bias_v[i - 1, pl.ds(j * 16, 16)]
                      for j in range(_D // 16)]

                def num_step(bb, c, i=i, wr=wr, br=br):
                    xrow = xnum_v[pl.ds((s * _BSUB + bb) * 16, 16)]
                    xv = jnp.full((16,), xrow[i - 1], jnp.float32)
                    for j in range(_D // 16):
                        combuf[bb, i, pl.ds(j * 16, 16)] = wr[j] * xv + br[j]
                    return c
            lax.fori_loop(0, _BSUB, num_step, 0)

        for c in copies:
            c.wait()

        for j in range(_D // 16):
            sl = pl.ds(j * 16, 16)
            br = [bias_v[_INUM + f, sl] for f in range(_CATS)]

            def bias_step(bb, c, br=br, sl=sl):
                for f in range(_CATS):
                    combuf[bb, _NUMROWS + f, sl] = (
                        catstage[bb * _CATS + f, sl] + br[f])
                return c
            lax.fori_loop(0, _BSUB, bias_step, 0)

        pltpu.sync_copy(combuf, out_hbm.at[pl.ds(bs, _BSUB)])
        return carry

    lax.fori_loop(0, _NSUB, sub_chunk, 0)


@jax.jit
def kernel(x_num, x_cat, cls_num_weight, cat_table, bias):
    offs = jnp.tile(jnp.arange(_CATS, dtype=jnp.int32) * _NCAT, _BSUB)
    xnum_flat = jnp.pad(x_num, ((0, 0), (0, 16 - _INUM))).reshape(-1)
    xcat_flat = x_cat.astype(jnp.int32).reshape(-1)

    mesh = plsc.VectorSubcoreMesh(core_axis_name="c", subcore_axis_name="s")
    k = pl.kernel(
        _sc_body,
        out_type=jax.ShapeDtypeStruct((_B, _TOK, _D), jnp.float32),
        mesh=mesh,
        compiler_params=pltpu.CompilerParams(use_tc_tiling_on_sc=False),
        scratch_types=[
            pltpu.VMEM((_BPW * 16,), jnp.float32),
            pltpu.VMEM((_NIDX,), jnp.int32),
            pltpu.VMEM((_NIDX,), jnp.int32),
            pltpu.VMEM((_NIDX,), jnp.int32),
            pltpu.VMEM((_NUMROWS, _D), jnp.float32),
            pltpu.VMEM((_INUM + _CATS, _D), jnp.float32),
            pltpu.VMEM((_NIDX, _D), jnp.float32),
            pltpu.VMEM((_BSUB, _TOK, _D), jnp.float32),
            pltpu.SemaphoreType.DMA,
        ],
    )
    return k(xnum_flat, xcat_flat, cls_num_weight, cat_table, bias, offs)


# speedup vs baseline: 2.5954x; 1.0555x over previous
import functools

import jax
import jax.numpy as jnp
from jax import lax
from jax.experimental import pallas as pl
from jax.experimental.pallas import tpu as pltpu
from jax.experimental.pallas import tpu_sc as plsc

_CATS = 26
_NCAT = 1000
_INUM = 13
_D = 64
_DP = 128
_B = 4096
_NC = 2
_NS = 16
_NW = _NC * _NS
_BPW = _B // _NW
_BSUB = 8
_NSUB = _BPW // _BSUB
_NIDX = _BSUB * _CATS
_PIECE = 104
_NPIECE = _NIDX // _PIECE
_NUMROWS = 1 + _INUM
_TOK = _NUMROWS + _CATS


def _sc_body(xnum_hbm, xcat_hbm, w_hbm, table_hbm, bias_hbm, offs_hbm,
             out_hbm,
             xnum_v, xcat_v, offs_v, idx_v, w_v, bias_v, catstage, combuf,
             gsem):
    wid = lax.axis_index("s") * _NC + lax.axis_index("c")
    b0 = wid * _BPW

    pltpu.sync_copy(w_hbm, w_v)
    pltpu.sync_copy(bias_hbm, bias_v)
    pltpu.sync_copy(offs_hbm, offs_v)
    pltpu.sync_copy(xnum_hbm.at[pl.ds(b0 * 16, _BPW * 16)], xnum_v)

    def sub_chunk(s, carry):
        bs = b0 + s * _BSUB

        pltpu.sync_copy(xcat_hbm.at[pl.ds(bs * _CATS, _NIDX)], xcat_v)

        def idx_step(j, c):
            sl = pl.ds(j * 16, 16)
            idx_v[sl] = xcat_v[sl] + offs_v[sl]
            return c
        lax.fori_loop(0, _NIDX // 16, idx_step, 0)

        copies = [
            pltpu.async_copy(
                table_hbm.at[idx_v.at[pl.ds(p * _PIECE, _PIECE)]],
                catstage.at[pl.ds(p * _PIECE, _PIECE), :],
                gsem)
            for p in range(_NPIECE)
        ]

        for i in range(_NUMROWS):
            wr = [w_v[i, pl.ds(j * 16, 16)] for j in range(_D // 16)]
            if i == 0:
                def num_step(bb, c, wr=wr):
                    for j in range(_D // 16):
                        combuf[bb, 0, pl.ds(j * 16, 16)] = wr[j]
                    return c
            else:
                br = [bias_v[i - 1, pl.ds(j * 16, 16)]
                      for j in range(_D // 16)]

                def num_step(bb, c, i=i, wr=wr, br=br):
                    xrow = xnum_v[pl.ds((s * _BSUB + bb) * 16, 16)]
                    xv = jnp.full((16,), xrow[i - 1], jnp.float32)
                    for j in range(_D // 16):
                        combuf[bb, i, pl.ds(j * 16, 16)] = wr[j] * xv + br[j]
                    return c
            lax.fori_loop(0, _BSUB, num_step, 0)

        for c in copies:
            c.wait()

        for j in range(_D // 16):
            sl = pl.ds(j * 16, 16)
            br = [bias_v[_INUM + f, sl] for f in range(_CATS)]

            def bias_step(bb, c, br=br, sl=sl):
                for f in range(_CATS):
                    combuf[bb, _NUMROWS + f, sl] = (
                        catstage[bb * _CATS + f, sl] + br[f])
                return c
            lax.fori_loop(0, _BSUB, bias_step, 0)

        pltpu.sync_copy(combuf, out_hbm.at[pl.ds(bs, _BSUB)])
        return carry

    lax.fori_loop(0, _NSUB, sub_chunk, 0)


@jax.jit
def kernel(x_num, x_cat, cls_num_weight, cat_table, bias):
    offs = jnp.tile(jnp.arange(_CATS, dtype=jnp.int32) * _NCAT, _BSUB)
    xnum_flat = jnp.pad(x_num, ((0, 0), (0, 16 - _INUM))).reshape(-1)
    xcat_flat = x_cat.astype(jnp.int32).reshape(-1)
    table_pad = jnp.pad(cat_table, ((0, 0), (0, _DP - _D)))

    mesh = plsc.VectorSubcoreMesh(core_axis_name="c", subcore_axis_name="s")
    k = pl.kernel(
        _sc_body,
        out_type=jax.ShapeDtypeStruct((_B, _TOK, _D), jnp.float32),
        mesh=mesh,
        compiler_params=pltpu.CompilerParams(use_tc_tiling_on_sc=True),
        scratch_types=[
            pltpu.VMEM((_BPW * 16,), jnp.float32),
            pltpu.VMEM((_NIDX,), jnp.int32),
            pltpu.VMEM((_NIDX,), jnp.int32),
            pltpu.VMEM((_NIDX,), jnp.int32),
            pltpu.VMEM((_NUMROWS, _D), jnp.float32),
            pltpu.VMEM((_INUM + _CATS, _D), jnp.float32),
            pltpu.VMEM((_NIDX, _DP), jnp.float32),
            pltpu.VMEM((_BSUB, _TOK, _D), jnp.float32),
            pltpu.SemaphoreType.DMA,
        ],
    )
    return k(xnum_flat, xcat_flat, cls_num_weight, table_pad, bias, offs)
